# speedup vs baseline: 14.5204x; 14.5204x over previous
import functools

import jax
import jax.numpy as jnp
from jax import lax
from jax.experimental import pallas as pl
from jax.experimental.pallas import tpu as pltpu
from jax.experimental.pallas import tpu_sc as plsc

N = 10000
E = 640000
H = 128
IN_PAD = 8

NC = 2
NS = 16
NW = NC * NS
NP = 10240
RPS = NP // NS

B = 128
EPW = 19968
CH = EPW // B
TAIL = (E - NW * EPW) // B

RB = 1024
GRID = NP // RB


def _sc_mesh():
    return plsc.VectorSubcoreMesh(core_axis_name="c", subcore_axis_name="s")


def _deg_kernel(dst, zeros1):
    @functools.partial(
        pl.kernel,
        out_type=jax.ShapeDtypeStruct((NC * NP,), jnp.float32),
        mesh=_sc_mesh(),
        scratch_types=[
            pltpu.VMEM((B,), jnp.int32),
            pltpu.VMEM((B,), jnp.float32),
            pltpu.VMEM_SHARED((NP,), jnp.float32),
        ],
    )
    def k(dst_hbm, z_hbm, degp_hbm, dst_v, ones_v, acc):
        c = lax.axis_index("c")
        s = lax.axis_index("s")
        wid = c * NS + s
        r0 = s * RPS
        for i in range(B // 16):
            ones_v[pl.ds(i * 16, 16)] = jnp.ones((16,), jnp.float32)
        pltpu.sync_copy(z_hbm.at[pl.ds(r0, RPS)], acc.at[pl.ds(r0, RPS)])
        plsc.subcore_barrier()

        def chunk(base):
            pltpu.sync_copy(dst_hbm.at[pl.ds(base, B)], dst_v)
            pltpu.sync_copy(ones_v, acc.at[dst_v], add=True)

        ebase = wid * EPW

        def body(i, carry):
            chunk(ebase + i * B)
            return carry

        lax.fori_loop(0, CH, body, 0)

        @pl.when(wid < TAIL)
        def _():
            chunk(NW * EPW + wid * B)

        plsc.subcore_barrier()
        pltpu.sync_copy(acc.at[pl.ds(r0, RPS)],
                        degp_hbm.at[pl.ds(c * NP + r0, RPS)])

    return k(dst, zeros1)


def _agg_kernel(y, src, dst, zeros2):
    @functools.partial(
        pl.kernel,
        out_type=jax.ShapeDtypeStruct((NC * NP, H), jnp.float32),
        mesh=_sc_mesh(),
        scratch_types=[
            pltpu.VMEM((B,), jnp.int32),
            pltpu.VMEM((B,), jnp.int32),
            pltpu.VMEM((B, H), jnp.float32),
            pltpu.VMEM_SHARED((NP, H), jnp.float32),
            pltpu.SemaphoreType.DMA,
        ],
    )
    def k(y_hbm, src_hbm, dst_hbm, z_hbm, part_hbm,
          src_v, dst_v, rows_v, acc, sem):
        c = lax.axis_index("c")
        s = lax.axis_index("s")
        wid = c * NS + s
        r0 = s * RPS
        pltpu.sync_copy(z_hbm.at[pl.ds(r0, RPS)], acc.at[pl.ds(r0, RPS)])
        plsc.subcore_barrier()

        def chunk(base):
            pltpu.sync_copy(src_hbm.at[pl.ds(base, B)], src_v)
            pltpu.sync_copy(dst_hbm.at[pl.ds(base, B)], dst_v)
            pltpu.async_copy(y_hbm.at[src_v], rows_v, sem).wait()
            pltpu.sync_copy(rows_v, acc.at[dst_v], add=True)

        ebase = wid * EPW

        def body(i, carry):
            chunk(ebase + i * B)
            return carry

        lax.fori_loop(0, CH, body, 0)

        @pl.when(wid < TAIL)
        def _():
            chunk(NW * EPW + wid * B)

        plsc.subcore_barrier()
        pltpu.sync_copy(acc.at[pl.ds(r0, RPS)],
                        part_hbm.at[pl.ds(c * NP + r0, RPS)])

    return k(y, src, dst, zeros2)


def _prep_kernel(xp, w1p, deg0, deg1):
    def body(x_ref, w_ref, d0_ref, d1_ref, y_ref, dinv_ref):
        deg = d0_ref[...] + d1_ref[...] + 1.0
        dinv = lax.rsqrt(deg)
        xw = jnp.dot(x_ref[...], w_ref[...],
                     preferred_element_type=jnp.float32)
        y_ref[...] = xw * dinv
        dinv_ref[...] = dinv

    return pl.pallas_call(
        body,
        grid=(GRID,),
        in_specs=[
            pl.BlockSpec((RB, IN_PAD), lambda i: (i, 0)),
            pl.BlockSpec((IN_PAD, H), lambda i: (0, 0)),
            pl.BlockSpec((RB, 1), lambda i: (i, 0)),
            pl.BlockSpec((RB, 1), lambda i: (i, 0)),
        ],
        out_specs=[
            pl.BlockSpec((RB, H), lambda i: (i, 0)),
            pl.BlockSpec((RB, 1), lambda i: (i, 0)),
        ],
        out_shape=[
            jax.ShapeDtypeStruct((NP, H), jnp.float32),
            jax.ShapeDtypeStruct((NP, 1), jnp.float32),
        ],
    )(xp, w1p, deg0, deg1)


def _combine_kernel(p0, p1, y, dinv, b, w_next):
    def body(p0_ref, p1_ref, y_ref, dinv_ref, b_ref, w_ref, out_ref):
        dinv = dinv_ref[...]
        h = (p0_ref[...] + p1_ref[...] + y_ref[...]) * dinv + b_ref[...]
        h = jnp.maximum(h, 0.0)
        out_ref[...] = jnp.dot(h, w_ref[...],
                               preferred_element_type=jnp.float32) * dinv

    return pl.pallas_call(
        body,
        grid=(GRID,),
        in_specs=[
            pl.BlockSpec((RB, H), lambda i: (i, 0)),
            pl.BlockSpec((RB, H), lambda i: (i, 0)),
            pl.BlockSpec((RB, H), lambda i: (i, 0)),
            pl.BlockSpec((RB, 1), lambda i: (i, 0)),
            pl.BlockSpec((1, H), lambda i: (0, 0)),
            pl.BlockSpec((H, H), lambda i: (0, 0)),
        ],
        out_specs=pl.BlockSpec((RB, H), lambda i: (i, 0)),
        out_shape=jax.ShapeDtypeStruct((NP, H), jnp.float32),
    )(p0, p1, y, dinv, b, w_next)


def _final_kernel(p0, p1, y, dinv, b3, q1, qb1, q2, qb2, q3, qb3):
    def body(p0_ref, p1_ref, y_ref, dinv_ref, b_ref,
             q1_ref, qb1_ref, q2_ref, qb2_ref, q3_ref, qb3_ref, out_ref):
        dinv = dinv_ref[...]
        h = (p0_ref[...] + p1_ref[...] + y_ref[...]) * dinv + b_ref[...]
        h = jnp.maximum(h, 0.0)
        t = jnp.dot(h, q1_ref[...], preferred_element_type=jnp.float32)
        t = t + qb1_ref[...]
        t = jnp.where(t >= 0.0, t, 0.01 * t)
        t = jnp.dot(t, q2_ref[...], preferred_element_type=jnp.float32)
        t = t + qb2_ref[...]
        t = jnp.where(t >= 0.0, t, 0.01 * t)
        t = jnp.dot(t, q3_ref[...], preferred_element_type=jnp.float32)
        out_ref[...] = t + qb3_ref[...]

    return pl.pallas_call(
        body,
        grid=(GRID,),
        in_specs=[
            pl.BlockSpec((RB, H), lambda i: (i, 0)),
            pl.BlockSpec((RB, H), lambda i: (i, 0)),
            pl.BlockSpec((RB, H), lambda i: (i, 0)),
            pl.BlockSpec((RB, 1), lambda i: (i, 0)),
            pl.BlockSpec((1, H), lambda i: (0, 0)),
            pl.BlockSpec((H, H), lambda i: (0, 0)),
            pl.BlockSpec((1, H), lambda i: (0, 0)),
            pl.BlockSpec((H, H), lambda i: (0, 0)),
            pl.BlockSpec((1, H), lambda i: (0, 0)),
            pl.BlockSpec((H, 1), lambda i: (0, 0)),
            pl.BlockSpec((1, 1), lambda i: (0, 0)),
        ],
        out_specs=pl.BlockSpec((RB, 1), lambda i: (i, 0)),
        out_shape=jax.ShapeDtypeStruct((NP, 1), jnp.float32),
    )(p0, p1, y, dinv, b3, q1, qb1, q2, qb2, q3, qb3)


def kernel(x, edge_index, W1, b1, W2, b2, W3, b3, Q1, qb1, Q2, qb2, Q3, qb3):
    src = edge_index[0]
    dst = edge_index[1]

    xp = jnp.zeros((NP, IN_PAD), jnp.float32).at[:N, :x.shape[1]].set(x)
    w1p = jnp.zeros((IN_PAD, H), jnp.float32).at[:W1.shape[0], :].set(W1)
    zeros1 = jnp.zeros((NP,), jnp.float32)
    zeros2 = jnp.zeros((NP, H), jnp.float32)

    degp = _deg_kernel(dst, zeros1)
    deg0 = degp[:NP].reshape(NP, 1)
    deg1 = degp[NP:].reshape(NP, 1)

    y1, dinv = _prep_kernel(xp, w1p, deg0, deg1)

    p = _agg_kernel(y1, src, dst, zeros2)
    y2 = _combine_kernel(p[:NP], p[NP:], y1, dinv, b1.reshape(1, H), W2)

    p = _agg_kernel(y2, src, dst, zeros2)
    y3 = _combine_kernel(p[:NP], p[NP:], y2, dinv, b2.reshape(1, H), W3)

    p = _agg_kernel(y3, src, dst, zeros2)
    out = _final_kernel(p[:NP], p[NP:], y3, dinv, b3.reshape(1, H),
                        Q1, qb1.reshape(1, H), Q2, qb2.reshape(1, H),
                        Q3, qb3.reshape(1, 1))
    return out[:N]
